# baseline (speedup 1.0000x reference)
import functools

import jax
import jax.numpy as jnp
from jax import lax
from jax.experimental import pallas as pl
from jax.experimental.pallas import tpu as pltpu
from jax.experimental.pallas import tpu_sc as plsc

_NBUF = 2


def _gather_kernel(b, f, d, chunk):
    n_rows = b * f
    fc_per_w = f // 2
    mesh = plsc.VectorSubcoreMesh(core_axis_name="c", subcore_axis_name="s")

    @functools.partial(
        pl.kernel,
        mesh=mesh,
        out_type=jax.ShapeDtypeStruct((n_rows, d), jnp.float32),
        scratch_types=[
            pltpu.VMEM((fc_per_w, chunk), jnp.int32),
            pltpu.VMEM((fc_per_w, chunk), jnp.int32),
            [pltpu.VMEM((chunk, d), jnp.float32) for _ in range(_NBUF)],
            [pltpu.SemaphoreType.DMA for _ in range(_NBUF)],
            [pltpu.SemaphoreType.DMA for _ in range(_NBUF)],
        ],
        compiler_params=pltpu.CompilerParams(use_tc_tiling_on_sc=False),
    )
    def k(xt_hbm, table_hbm, out_hbm, idx_all, dest_all, rows, sem_g, sem_o):
        cid = lax.axis_index("c")
        sid = lax.axis_index("s")
        fc0 = cid * fc_per_w
        b0 = sid * chunk

        pltpu.sync_copy(
            xt_hbm.at[pl.ds(fc0, fc_per_w), pl.ds(b0, chunk)], idx_all
        )

        lane = lax.iota(jnp.int32, 16) * f
        for j in range(fc_per_w):
            fc = fc0 + j
            for m in range(chunk // 16):
                dest_all[j, pl.ds(m * 16, 16)] = lane + ((b0 + m * 16) * f + fc)

        gathers = {}
        stores = {}

        def start_store(j):
            r = j % _NBUF
            gathers[j].wait()
            stores[j] = pltpu.async_copy(
                rows[r], out_hbm.at[dest_all.at[j]], sem_o[r]
            )

        for i in range(fc_per_w):
            r = i % _NBUF
            if i >= _NBUF:
                stores[i - _NBUF].wait()
            gathers[i] = pltpu.async_copy(
                table_hbm.at[idx_all.at[i]], rows[r], sem_g[r]
            )
            if i >= 1:
                start_store(i - 1)
        start_store(fc_per_w - 1)
        for j in range(max(0, fc_per_w - _NBUF + 1), fc_per_w):
            stores[j].wait()

    return k


def kernel(x, table):
    b, f = x.shape
    v, d = table.shape
    out = _gather_kernel(b, f, d, chunk=1024)(x.T, table)
    return out.reshape(b, f * d)
